# Initial kernel scaffold; baseline (speedup 1.0000x reference)
import jax
import jax.numpy as jnp
from jax.experimental import pallas as pl


def kernel(tokens, table):
    raise NotImplementedError("write your pallas kernel here")


# speedup vs baseline: 3.6704x; 3.6704x over previous
import functools

import jax
import jax.numpy as jnp
from jax import lax
from jax.experimental import pallas as pl
from jax.experimental.pallas import tpu as pltpu
from jax.experimental.pallas import tpu_sc as plsc

VOCAB = 100000
EMB = 64
SCALE = 8.0

NC = 2
NS = 16
NW = NC * NS

GROUP = 128
K = 4


def _scale_body(t_ref, o_ref):
    o_ref[...] = t_ref[...] * SCALE


def _scale_table(table):
    blk = 4000
    return pl.pallas_call(
        _scale_body,
        grid=(VOCAB // blk,),
        in_specs=[pl.BlockSpec((blk, EMB), lambda i: (i, 0))],
        out_specs=pl.BlockSpec((blk, EMB), lambda i: (i, 0)),
        out_shape=jax.ShapeDtypeStruct((VOCAB, EMB), jnp.float32),
    )(table)


@functools.partial(jax.jit, static_argnums=(2,))
def _gather_call(tok, tab, rows):
    rows_per_w = rows // NW
    chunks = rows_per_w // K
    mesh = plsc.VectorSubcoreMesh(core_axis_name="c", subcore_axis_name="s")

    @functools.partial(
        pl.kernel,
        mesh=mesh,
        compiler_params=pltpu.CompilerParams(use_tc_tiling_on_sc=False),
        out_type=jax.ShapeDtypeStruct((rows, GROUP, EMB), jnp.float32),
        scratch_types=[
            pltpu.VMEM((K, GROUP), jnp.int32),
            pltpu.VMEM((K, GROUP, EMB), jnp.float32),
            pltpu.SemaphoreType.DMA,
        ],
    )
    def gather(tok_hbm, tab_hbm, out_hbm, idx_v, rows_v, sem):
        wid = lax.axis_index("s") * NC + lax.axis_index("c")
        base = wid * rows_per_w

        def chunk_body(c, carry):
            row0 = base + c * K
            pltpu.sync_copy(tok_hbm.at[pl.ds(row0, K)], idx_v)
            cps = [
                pltpu.async_copy(tab_hbm.at[idx_v.at[j]], rows_v.at[j], sem)
                for j in range(K)
            ]
            for cp in cps:
                cp.wait()
            pltpu.sync_copy(rows_v, out_hbm.at[pl.ds(row0, K)])
            return carry

        lax.fori_loop(0, chunks, chunk_body, 0)

    return gather(tok, tab)


def kernel(tokens, table):
    B, L = tokens.shape
    n = B * L
    rows = n // GROUP
    tok = tokens.reshape(rows, GROUP).astype(jnp.int32)
    tab = _scale_table(table)
    out = _gather_call(tok, tab, rows)
    return out.reshape(B, L, EMB)
